# baseline (speedup 1.0000x reference)
import jax
import jax.numpy as jnp
from jax.experimental import pallas as pl
from jax.experimental.pallas import tpu as pltpu

N_L = 2048
N_P = 8192
DIM = 256
BP = 1024
C = N_P // BP
LOG2E = 1.4426950408889634
SHIFT = 8.0 * LOG2E
LAM = 1000.0


def _body(sL_ref, A_ref, B_ref, sP_ref,
          Wq_ref, bq_ref, Wk_ref, bk_ref, Wv_ref, bv_ref,
          out_ref, q_ref, acc_ref, z_ref):
    j = pl.program_id(0)

    @pl.when(j == 0)
    def _init():
        q_ref[...] = ((jnp.dot(sL_ref[...].astype(jnp.bfloat16),
                               Wq_ref[...].astype(jnp.bfloat16).T,
                               preferred_element_type=jnp.float32)
                       + bq_ref[...]) * (0.125 * LOG2E)).astype(jnp.bfloat16)
        acc_ref[...] = jnp.zeros_like(acc_ref)
        z_ref[0] = 0.0

    sP16 = sP_ref[...].astype(jnp.bfloat16)
    k = (jnp.dot(sP16, Wk_ref[...].astype(jnp.bfloat16).T,
                 preferred_element_type=jnp.float32)
         + bk_ref[...]).astype(jnp.bfloat16)
    v = (jnp.dot(sP16, Wv_ref[...].astype(jnp.bfloat16).T,
                 preferred_element_type=jnp.float32)
         + bv_ref[...]).astype(jnp.bfloat16)
    t = jnp.dot(A_ref[...].astype(jnp.bfloat16),
                B_ref[...].astype(jnp.bfloat16).T,
                preferred_element_type=jnp.float32)
    s = jnp.dot(q_ref[...], k.T, preferred_element_type=jnp.float32)
    arg = s - LAM * jnp.maximum(t, SHIFT / LAM)
    e = jnp.exp2(arg)
    z_ref[0] += jnp.sum(e)
    acc_ref[...] += jnp.dot(e.astype(jnp.bfloat16), v,
                            preferred_element_type=jnp.float32)

    @pl.when(j == C - 1)
    def _final():
        Z = z_ref[0]
        Zs = jnp.where(Z > 0.0, Z, 1.0)
        out_ref[...] = sL_ref[...] + acc_ref[...] * (1.0 / Zs)


def _attend(s_L, A, B, s_P, Wq, bq, Wk, bk, Wv, bv, interpret=False):
    grid = (C,)
    res = lambda i: (0, 0)
    col = lambda i: (i, 0)
    out = pl.pallas_call(
        _body,
        grid=grid,
        in_specs=[
            pl.BlockSpec((N_L, DIM), res),
            pl.BlockSpec((N_L, 8), res),
            pl.BlockSpec((BP, 8), col),
            pl.BlockSpec((BP, DIM), col),
            pl.BlockSpec((DIM, DIM), res),
            pl.BlockSpec((1, DIM), res),
            pl.BlockSpec((DIM, DIM), res),
            pl.BlockSpec((1, DIM), res),
            pl.BlockSpec((DIM, DIM), res),
            pl.BlockSpec((1, DIM), res),
        ],
        out_specs=pl.BlockSpec((N_L, DIM), res),
        out_shape=jax.ShapeDtypeStruct((N_L, DIM), jnp.float32),
        scratch_shapes=[
            pltpu.VMEM((N_L, DIM), jnp.bfloat16),
            pltpu.VMEM((N_L, DIM), jnp.float32),
            pltpu.SMEM((1,), jnp.float32),
        ],
        interpret=interpret,
    )(s_L, A, B, s_P, Wq, bq, Wk, bk, Wv, bv)
    return out


def kernel(s_L, v_L, pos_L, s_P, v_P, pos_P, Wq, bq, Wk, bk, Wv, bv):
    nL = jnp.sum(pos_L * pos_L, axis=1, keepdims=True)
    nP = jnp.sum(pos_P * pos_P, axis=1, keepdims=True)
    nL_hi = jnp.floor(nL / 256.0) * 256.0
    nL_lo = nL - nL_hi
    nP_hi = jnp.floor(nP / 256.0) * 256.0
    nP_lo = nP - nP_hi
    oneL = jnp.ones_like(nL)
    oneP = jnp.ones_like(nP)
    zL = jnp.zeros((N_L, 1), jnp.float32)
    zP = jnp.zeros((N_P, 1), jnp.float32)
    A = jnp.concatenate([pos_L, nL_hi, nL_lo, oneL, oneL, zL], axis=1)
    B = jnp.concatenate([-2.0 * pos_P, oneP, oneP, nP_hi, nP_lo - 100.0, zP],
                        axis=1)
    s_L_out = _attend(s_L, A, B, s_P,
                      Wq, bq.reshape(1, DIM),
                      Wk, bk.reshape(1, DIM),
                      Wv, bv.reshape(1, DIM))
    return (s_L_out, v_L)


# speedup vs baseline: 1.2638x; 1.2638x over previous
import jax
import jax.numpy as jnp
from jax.experimental import pallas as pl
from jax.experimental.pallas import tpu as pltpu

N_L = 2048
N_P = 8192
DIM = 256
BP = 2048
C = N_P // BP
LOG2E = 1.4426950408889634
SHIFT = 8.0 * LOG2E
LAM = 1000.0
NSUB = 4
SUB = BP // NSUB


def _body(sL_ref, A_ref, B_ref, sP_ref,
          Wq_ref, bq_ref, Wk_ref, bk_ref, Wv_ref, bv_ref,
          out_ref, q_ref, acc_ref, z_ref):
    j = pl.program_id(0)

    @pl.when(j == 0)
    def _init():
        q_ref[...] = ((jnp.dot(sL_ref[...].astype(jnp.bfloat16),
                               Wq_ref[...].astype(jnp.bfloat16).T,
                               preferred_element_type=jnp.float32)
                       + bq_ref[...]) * (0.125 * LOG2E)).astype(jnp.bfloat16)
        acc_ref[...] = jnp.zeros_like(acc_ref)
        z_ref[0] = 0.0

    sP16 = sP_ref[...].astype(jnp.bfloat16)
    k = (jnp.dot(sP16, Wk_ref[...].astype(jnp.bfloat16).T,
                 preferred_element_type=jnp.float32)
         + bk_ref[...]).astype(jnp.bfloat16)
    v = (jnp.dot(sP16, Wv_ref[...].astype(jnp.bfloat16).T,
                 preferred_element_type=jnp.float32)
         + bv_ref[...]).astype(jnp.bfloat16)
    A16 = A_ref[...].astype(jnp.bfloat16)
    B16 = B_ref[...].astype(jnp.bfloat16)
    q = q_ref[...]
    accv = acc_ref[...]
    z = 0.0
    for c in range(NSUB):
        lo = c * SUB
        t = jnp.dot(A16, B16[lo:lo + SUB, :].T,
                    preferred_element_type=jnp.float32)
        s = jnp.dot(q, k[lo:lo + SUB, :].T, preferred_element_type=jnp.float32)
        arg = s - LAM * jnp.maximum(t, SHIFT / LAM)
        e = jnp.exp2(arg)
        z += jnp.sum(e)
        accv += jnp.dot(e.astype(jnp.bfloat16), v[lo:lo + SUB, :],
                        preferred_element_type=jnp.float32)
    z_ref[0] += z
    acc_ref[...] = accv

    @pl.when(j == C - 1)
    def _final():
        Z = z_ref[0]
        Zs = jnp.where(Z > 0.0, Z, 1.0)
        out_ref[...] = sL_ref[...] + acc_ref[...] * (1.0 / Zs)


def _attend(s_L, A, B, s_P, Wq, bq, Wk, bk, Wv, bv, interpret=False):
    grid = (C,)
    res = lambda i: (0, 0)
    col = lambda i: (i, 0)
    out = pl.pallas_call(
        _body,
        grid=grid,
        in_specs=[
            pl.BlockSpec((N_L, DIM), res),
            pl.BlockSpec((N_L, 8), res),
            pl.BlockSpec((BP, 8), col),
            pl.BlockSpec((BP, DIM), col),
            pl.BlockSpec((DIM, DIM), res),
            pl.BlockSpec((1, DIM), res),
            pl.BlockSpec((DIM, DIM), res),
            pl.BlockSpec((1, DIM), res),
            pl.BlockSpec((DIM, DIM), res),
            pl.BlockSpec((1, DIM), res),
        ],
        out_specs=pl.BlockSpec((N_L, DIM), res),
        out_shape=jax.ShapeDtypeStruct((N_L, DIM), jnp.float32),
        scratch_shapes=[
            pltpu.VMEM((N_L, DIM), jnp.bfloat16),
            pltpu.VMEM((N_L, DIM), jnp.float32),
            pltpu.SMEM((1,), jnp.float32),
        ],
        interpret=interpret,
    )(s_L, A, B, s_P, Wq, bq, Wk, bk, Wv, bv)
    return out


def kernel(s_L, v_L, pos_L, s_P, v_P, pos_P, Wq, bq, Wk, bk, Wv, bv):
    nL = jnp.sum(pos_L * pos_L, axis=1, keepdims=True)
    nP = jnp.sum(pos_P * pos_P, axis=1, keepdims=True)
    nL_hi = jnp.floor(nL / 256.0) * 256.0
    nL_lo = nL - nL_hi
    nP_hi = jnp.floor(nP / 256.0) * 256.0
    nP_lo = nP - nP_hi
    oneL = jnp.ones_like(nL)
    oneP = jnp.ones_like(nP)
    zL = jnp.zeros((N_L, 1), jnp.float32)
    zP = jnp.zeros((N_P, 1), jnp.float32)
    A = jnp.concatenate([pos_L, nL_hi, nL_lo, oneL, oneL, zL], axis=1)
    B = jnp.concatenate([-2.0 * pos_P, oneP, oneP, nP_hi, nP_lo - 100.0, zP],
                        axis=1)
    s_L_out = _attend(s_L, A, B, s_P,
                      Wq, bq.reshape(1, DIM),
                      Wk, bk.reshape(1, DIM),
                      Wv, bv.reshape(1, DIM))
    return (s_L_out, v_L)


# speedup vs baseline: 1.2652x; 1.0011x over previous
import jax
import jax.numpy as jnp
from jax.experimental import pallas as pl
from jax.experimental.pallas import tpu as pltpu

N_L = 2048
N_P = 8192
DIM = 256
BP = 2048
C = N_P // BP
LOG2E = 1.4426950408889634
SHIFT = 8.0 * LOG2E
LAM = 1000.0
NSUB = 8
SUB = BP // NSUB


def _body(sL_ref, A_ref, B_ref, sP_ref,
          Wq_ref, bq_ref, Wk_ref, bk_ref, Wv_ref, bv_ref,
          out_ref, q_ref, acc_ref, z_ref):
    j = pl.program_id(0)

    @pl.when(j == 0)
    def _init():
        q_ref[...] = ((jnp.dot(sL_ref[...].astype(jnp.bfloat16),
                               Wq_ref[...].astype(jnp.bfloat16).T,
                               preferred_element_type=jnp.float32)
                       + bq_ref[...]) * (0.125 * LOG2E)).astype(jnp.bfloat16)
        acc_ref[...] = jnp.zeros_like(acc_ref)
        z_ref[0] = 0.0

    sP16 = sP_ref[...].astype(jnp.bfloat16)
    k = (jnp.dot(sP16, Wk_ref[...].astype(jnp.bfloat16).T,
                 preferred_element_type=jnp.float32)
         + bk_ref[...]).astype(jnp.bfloat16)
    v = (jnp.dot(sP16, Wv_ref[...].astype(jnp.bfloat16).T,
                 preferred_element_type=jnp.float32)
         + bv_ref[...]).astype(jnp.bfloat16)
    A16 = A_ref[...].astype(jnp.bfloat16)
    B16 = B_ref[...].astype(jnp.bfloat16)
    q = q_ref[...]
    accv = acc_ref[...]
    z = 0.0
    for c in range(NSUB):
        lo = c * SUB
        t = jnp.dot(A16, B16[lo:lo + SUB, :].T,
                    preferred_element_type=jnp.float32)
        s = jnp.dot(q, k[lo:lo + SUB, :].T, preferred_element_type=jnp.float32)
        arg = s - LAM * jnp.maximum(t, SHIFT / LAM)
        e = jnp.exp2(arg)
        z += jnp.sum(e)
        accv += jnp.dot(e.astype(jnp.bfloat16), v[lo:lo + SUB, :],
                        preferred_element_type=jnp.float32)
    z_ref[0] += z
    acc_ref[...] = accv

    @pl.when(j == C - 1)
    def _final():
        Z = z_ref[0]
        Zs = jnp.where(Z > 0.0, Z, 1.0)
        out_ref[...] = sL_ref[...] + acc_ref[...] * (1.0 / Zs)


def _attend(s_L, A, B, s_P, Wq, bq, Wk, bk, Wv, bv, interpret=False):
    grid = (C,)
    res = lambda i: (0, 0)
    col = lambda i: (i, 0)
    out = pl.pallas_call(
        _body,
        grid=grid,
        in_specs=[
            pl.BlockSpec((N_L, DIM), res),
            pl.BlockSpec((N_L, 8), res),
            pl.BlockSpec((BP, 8), col),
            pl.BlockSpec((BP, DIM), col),
            pl.BlockSpec((DIM, DIM), res),
            pl.BlockSpec((1, DIM), res),
            pl.BlockSpec((DIM, DIM), res),
            pl.BlockSpec((1, DIM), res),
            pl.BlockSpec((DIM, DIM), res),
            pl.BlockSpec((1, DIM), res),
        ],
        out_specs=pl.BlockSpec((N_L, DIM), res),
        out_shape=jax.ShapeDtypeStruct((N_L, DIM), jnp.float32),
        scratch_shapes=[
            pltpu.VMEM((N_L, DIM), jnp.bfloat16),
            pltpu.VMEM((N_L, DIM), jnp.float32),
            pltpu.SMEM((1,), jnp.float32),
        ],
        interpret=interpret,
    )(s_L, A, B, s_P, Wq, bq, Wk, bk, Wv, bv)
    return out


def kernel(s_L, v_L, pos_L, s_P, v_P, pos_P, Wq, bq, Wk, bk, Wv, bv):
    nL = jnp.sum(pos_L * pos_L, axis=1, keepdims=True)
    nP = jnp.sum(pos_P * pos_P, axis=1, keepdims=True)
    nL_hi = jnp.floor(nL / 256.0) * 256.0
    nL_lo = nL - nL_hi
    nP_hi = jnp.floor(nP / 256.0) * 256.0
    nP_lo = nP - nP_hi
    oneL = jnp.ones_like(nL)
    oneP = jnp.ones_like(nP)
    zL = jnp.zeros((N_L, 1), jnp.float32)
    zP = jnp.zeros((N_P, 1), jnp.float32)
    A = jnp.concatenate([pos_L, nL_hi, nL_lo, oneL, oneL, zL], axis=1)
    B = jnp.concatenate([-2.0 * pos_P, oneP, oneP, nP_hi, nP_lo - 100.0, zP],
                        axis=1)
    s_L_out = _attend(s_L, A, B, s_P,
                      Wq, bq.reshape(1, DIM),
                      Wk, bk.reshape(1, DIM),
                      Wv, bv.reshape(1, DIM))
    return (s_L_out, v_L)


# speedup vs baseline: 1.2817x; 1.0130x over previous
import jax
import jax.numpy as jnp
from jax.experimental import pallas as pl
from jax.experimental.pallas import tpu as pltpu

N_L = 2048
N_P = 8192
DIM = 256
BP = 8192
C = N_P // BP
LOG2E = 1.4426950408889634
SHIFT = 8.0 * LOG2E
LAM = 1000.0
NSUB = 16
SUB = BP // NSUB


def _body(sL_ref, A_ref, B_ref, sP_ref,
          Wq_ref, bq_ref, Wk_ref, bk_ref, Wv_ref, bv_ref,
          out_ref, q_ref, acc_ref, z_ref):
    j = pl.program_id(0)

    @pl.when(j == 0)
    def _init():
        q_ref[...] = ((jnp.dot(sL_ref[...].astype(jnp.bfloat16),
                               Wq_ref[...].astype(jnp.bfloat16).T,
                               preferred_element_type=jnp.float32)
                       + bq_ref[...]) * (0.125 * LOG2E)).astype(jnp.bfloat16)
        acc_ref[...] = jnp.zeros_like(acc_ref)
        z_ref[0] = 0.0

    sP16 = sP_ref[...].astype(jnp.bfloat16)
    k = (jnp.dot(sP16, Wk_ref[...].astype(jnp.bfloat16).T,
                 preferred_element_type=jnp.float32)
         + bk_ref[...]).astype(jnp.bfloat16)
    v = (jnp.dot(sP16, Wv_ref[...].astype(jnp.bfloat16).T,
                 preferred_element_type=jnp.float32)
         + bv_ref[...]).astype(jnp.bfloat16)
    A16 = A_ref[...].astype(jnp.bfloat16)
    B16 = B_ref[...].astype(jnp.bfloat16)
    q = q_ref[...]
    accv = acc_ref[...]
    z = 0.0
    for c in range(NSUB):
        lo = c * SUB
        t = jnp.dot(A16, B16[lo:lo + SUB, :].T,
                    preferred_element_type=jnp.float32)
        s = jnp.dot(q, k[lo:lo + SUB, :].T, preferred_element_type=jnp.float32)
        arg = s - LAM * jnp.maximum(t, SHIFT / LAM)
        e = jnp.exp2(arg)
        z += jnp.sum(e)
        accv += jnp.dot(e.astype(jnp.bfloat16), v[lo:lo + SUB, :],
                        preferred_element_type=jnp.float32)
    z_ref[0] += z
    acc_ref[...] = accv

    @pl.when(j == C - 1)
    def _final():
        Z = z_ref[0]
        Zs = jnp.where(Z > 0.0, Z, 1.0)
        out_ref[...] = sL_ref[...] + acc_ref[...] * (1.0 / Zs)


def _attend(s_L, A, B, s_P, Wq, bq, Wk, bk, Wv, bv, interpret=False):
    grid = (C,)
    res = lambda i: (0, 0)
    col = lambda i: (i, 0)
    out = pl.pallas_call(
        _body,
        grid=grid,
        in_specs=[
            pl.BlockSpec((N_L, DIM), res),
            pl.BlockSpec((N_L, 8), res),
            pl.BlockSpec((BP, 8), col),
            pl.BlockSpec((BP, DIM), col),
            pl.BlockSpec((DIM, DIM), res),
            pl.BlockSpec((1, DIM), res),
            pl.BlockSpec((DIM, DIM), res),
            pl.BlockSpec((1, DIM), res),
            pl.BlockSpec((DIM, DIM), res),
            pl.BlockSpec((1, DIM), res),
        ],
        out_specs=pl.BlockSpec((N_L, DIM), res),
        out_shape=jax.ShapeDtypeStruct((N_L, DIM), jnp.float32),
        scratch_shapes=[
            pltpu.VMEM((N_L, DIM), jnp.bfloat16),
            pltpu.VMEM((N_L, DIM), jnp.float32),
            pltpu.SMEM((1,), jnp.float32),
        ],
        interpret=interpret,
    )(s_L, A, B, s_P, Wq, bq, Wk, bk, Wv, bv)
    return out


def kernel(s_L, v_L, pos_L, s_P, v_P, pos_P, Wq, bq, Wk, bk, Wv, bv):
    nL = jnp.sum(pos_L * pos_L, axis=1, keepdims=True)
    nP = jnp.sum(pos_P * pos_P, axis=1, keepdims=True)
    nL_hi = jnp.floor(nL / 256.0) * 256.0
    nL_lo = nL - nL_hi
    nP_hi = jnp.floor(nP / 256.0) * 256.0
    nP_lo = nP - nP_hi
    oneL = jnp.ones_like(nL)
    oneP = jnp.ones_like(nP)
    zL = jnp.zeros((N_L, 1), jnp.float32)
    zP = jnp.zeros((N_P, 1), jnp.float32)
    A = jnp.concatenate([pos_L, nL_hi, nL_lo, oneL, oneL, zL], axis=1)
    B = jnp.concatenate([-2.0 * pos_P, oneP, oneP, nP_hi, nP_lo - 100.0, zP],
                        axis=1)
    s_L_out = _attend(s_L, A, B, s_P,
                      Wq, bq.reshape(1, DIM),
                      Wk, bk.reshape(1, DIM),
                      Wv, bv.reshape(1, DIM))
    return (s_L_out, v_L)


# speedup vs baseline: 1.2985x; 1.0131x over previous
import jax
import jax.numpy as jnp
from jax.experimental import pallas as pl
from jax.experimental.pallas import tpu as pltpu

N_L = 2048
N_P = 8192
DIM = 256
BP = 8192
C = N_P // BP
LOG2E = 1.4426950408889634
SHIFT = 8.0 * LOG2E
LAM = 1000.0
NSUB = 32
SUB = BP // NSUB


def _body(sL_ref, A_ref, B_ref, sP_ref,
          Wq_ref, bq_ref, Wk_ref, bk_ref, Wv_ref, bv_ref,
          out_ref, q_ref, acc_ref, z_ref):
    j = pl.program_id(0)

    @pl.when(j == 0)
    def _init():
        q_ref[...] = ((jnp.dot(sL_ref[...].astype(jnp.bfloat16),
                               Wq_ref[...].astype(jnp.bfloat16).T,
                               preferred_element_type=jnp.float32)
                       + bq_ref[...]) * (0.125 * LOG2E)).astype(jnp.bfloat16)
        acc_ref[...] = jnp.zeros_like(acc_ref)
        z_ref[0] = 0.0

    sP16 = sP_ref[...].astype(jnp.bfloat16)
    k = (jnp.dot(sP16, Wk_ref[...].astype(jnp.bfloat16).T,
                 preferred_element_type=jnp.float32)
         + bk_ref[...]).astype(jnp.bfloat16)
    v = (jnp.dot(sP16, Wv_ref[...].astype(jnp.bfloat16).T,
                 preferred_element_type=jnp.float32)
         + bv_ref[...]).astype(jnp.bfloat16)
    A16 = A_ref[...].astype(jnp.bfloat16)
    B16 = B_ref[...].astype(jnp.bfloat16)
    q = q_ref[...]
    accv = acc_ref[...]
    z = 0.0
    for c in range(NSUB):
        lo = c * SUB
        t = jnp.dot(A16, B16[lo:lo + SUB, :].T,
                    preferred_element_type=jnp.float32)
        s = jnp.dot(q, k[lo:lo + SUB, :].T, preferred_element_type=jnp.float32)
        arg = s - LAM * jnp.maximum(t, SHIFT / LAM)
        e = jnp.exp2(arg)
        z += jnp.sum(e)
        accv += jnp.dot(e.astype(jnp.bfloat16), v[lo:lo + SUB, :],
                        preferred_element_type=jnp.float32)
    z_ref[0] += z
    acc_ref[...] = accv

    @pl.when(j == C - 1)
    def _final():
        Z = z_ref[0]
        Zs = jnp.where(Z > 0.0, Z, 1.0)
        out_ref[...] = sL_ref[...] + acc_ref[...] * (1.0 / Zs)


def _attend(s_L, A, B, s_P, Wq, bq, Wk, bk, Wv, bv, interpret=False):
    grid = (C,)
    res = lambda i: (0, 0)
    col = lambda i: (i, 0)
    out = pl.pallas_call(
        _body,
        grid=grid,
        in_specs=[
            pl.BlockSpec((N_L, DIM), res),
            pl.BlockSpec((N_L, 8), res),
            pl.BlockSpec((BP, 8), col),
            pl.BlockSpec((BP, DIM), col),
            pl.BlockSpec((DIM, DIM), res),
            pl.BlockSpec((1, DIM), res),
            pl.BlockSpec((DIM, DIM), res),
            pl.BlockSpec((1, DIM), res),
            pl.BlockSpec((DIM, DIM), res),
            pl.BlockSpec((1, DIM), res),
        ],
        out_specs=pl.BlockSpec((N_L, DIM), res),
        out_shape=jax.ShapeDtypeStruct((N_L, DIM), jnp.float32),
        scratch_shapes=[
            pltpu.VMEM((N_L, DIM), jnp.bfloat16),
            pltpu.VMEM((N_L, DIM), jnp.float32),
            pltpu.SMEM((1,), jnp.float32),
        ],
        interpret=interpret,
    )(s_L, A, B, s_P, Wq, bq, Wk, bk, Wv, bv)
    return out


def kernel(s_L, v_L, pos_L, s_P, v_P, pos_P, Wq, bq, Wk, bk, Wv, bv):
    nL = jnp.sum(pos_L * pos_L, axis=1, keepdims=True)
    nP = jnp.sum(pos_P * pos_P, axis=1, keepdims=True)
    nL_hi = jnp.floor(nL / 256.0) * 256.0
    nL_lo = nL - nL_hi
    nP_hi = jnp.floor(nP / 256.0) * 256.0
    nP_lo = nP - nP_hi
    oneL = jnp.ones_like(nL)
    oneP = jnp.ones_like(nP)
    zL = jnp.zeros((N_L, 1), jnp.float32)
    zP = jnp.zeros((N_P, 1), jnp.float32)
    A = jnp.concatenate([pos_L, nL_hi, nL_lo, oneL, oneL, zL], axis=1)
    B = jnp.concatenate([-2.0 * pos_P, oneP, oneP, nP_hi, nP_lo - 100.0, zP],
                        axis=1)
    s_L_out = _attend(s_L, A, B, s_P,
                      Wq, bq.reshape(1, DIM),
                      Wk, bk.reshape(1, DIM),
                      Wv, bv.reshape(1, DIM))
    return (s_L_out, v_L)
